# Initial kernel scaffold; baseline (speedup 1.0000x reference)
import jax
import jax.numpy as jnp
from jax.experimental import pallas as pl


def kernel(imputs, table):
    raise NotImplementedError("write your pallas kernel here")


# speedup vs baseline: 1.5726x; 1.5726x over previous
import functools

import jax
import jax.numpy as jnp
from jax import lax
from jax.experimental import pallas as pl
from jax.experimental.pallas import tpu as pltpu
from jax.experimental.pallas import tpu_sc as plsc

NC = 2
NS = 16
NW = NC * NS

D = 64
CH = 128


@functools.partial(jax.jit, static_argnames=("n_rows",))
def _gather(idx_flat, table, *, n_rows):
    per_w = n_rows // NW
    n_ch = per_w // CH
    mesh = plsc.VectorSubcoreMesh(core_axis_name="c", subcore_axis_name="s")

    @functools.partial(
        pl.kernel,
        out_type=jax.ShapeDtypeStruct((n_rows, D), jnp.float32),
        mesh=mesh,
        scratch_types=[
            pltpu.VMEM((CH,), jnp.int32),
            pltpu.VMEM((CH, D), jnp.float32),
            pltpu.SemaphoreType.DMA,
        ],
        compiler_params=pltpu.CompilerParams(use_tc_tiling_on_sc=False),
    )
    def k(table_hbm, idx_hbm, out_hbm, idx_v, rows_v, sem):
        wid = lax.axis_index("s") * NC + lax.axis_index("c")
        w_base = wid * per_w

        def body(g, _):
            base = w_base + g * CH
            pltpu.sync_copy(idx_hbm.at[pl.ds(base, CH)], idx_v)
            pltpu.async_copy(table_hbm.at[idx_v], rows_v, sem).wait()
            pltpu.sync_copy(rows_v, out_hbm.at[pl.ds(base, CH)])
            return _

        lax.fori_loop(0, n_ch, body, 0)

    return k(table, idx_flat)


def kernel(imputs, table):
    b, h = imputs.shape
    idx_flat = imputs.reshape(b * h).astype(jnp.int32)
    out = _gather(idx_flat, table, n_rows=b * h)
    return out.reshape(b, h, D)


# speedup vs baseline: 1.8764x; 1.1931x over previous
import functools

import jax
import jax.numpy as jnp
from jax import lax
from jax.experimental import pallas as pl
from jax.experimental.pallas import tpu as pltpu
from jax.experimental.pallas import tpu_sc as plsc

NC = 2
NS = 16
NW = NC * NS

D = 64
CH = 128
NBUF = 8
LOOK = 4


@functools.partial(jax.jit, static_argnames=("n_rows",))
def _gather(idx3, table, *, n_rows):
    per_w = n_rows // NW
    n_ch = per_w // CH
    mesh = plsc.VectorSubcoreMesh(core_axis_name="c", subcore_axis_name="s")

    @functools.partial(
        pl.kernel,
        out_type=jax.ShapeDtypeStruct((n_rows, D), jnp.float32),
        mesh=mesh,
        scratch_types=[
            pltpu.VMEM((n_ch, CH), jnp.int32),
            [pltpu.VMEM((CH, D), jnp.float32) for _ in range(NBUF)],
            [pltpu.SemaphoreType.DMA for _ in range(NBUF)],
            [pltpu.SemaphoreType.DMA for _ in range(NBUF)],
        ],
        compiler_params=pltpu.CompilerParams(use_tc_tiling_on_sc=False),
    )
    def k(table_hbm, idx_hbm, out_hbm, idx_v, bufs, gsems, wsems):
        wid = lax.axis_index("s") * NC + lax.axis_index("c")
        w_base = wid * per_w

        pltpu.sync_copy(idx_hbm.at[wid], idx_v)

        def gather_chunk(g, b):
            return pltpu.make_async_copy(
                table_hbm.at[idx_v.at[g]], bufs[b], gsems[b])

        def write_chunk(g, b):
            return pltpu.make_async_copy(
                bufs[b], out_hbm.at[pl.ds(w_base + g * CH, CH)], wsems[b])

        for b in range(LOOK):
            gather_chunk(b, b).start()

        def outer(i, carry):
            so = i * NBUF
            for b in range(NBUF):
                g = so + b
                gather_chunk(g, b).wait()
                write_chunk(g, b).start()
                bn = (b + LOOK) % NBUF

                @pl.when(g + LOOK < n_ch)
                def _():
                    @pl.when(g + LOOK >= NBUF)
                    def _():
                        write_chunk(g + LOOK - NBUF, bn).wait()

                    gather_chunk(g + LOOK, bn).start()
            return carry

        lax.fori_loop(0, n_ch // NBUF, outer, 0)

        for j in range(NBUF - LOOK):
            g = n_ch - (NBUF - LOOK) + j
            write_chunk(g, g % NBUF).wait()

    return k(table, idx3)


def kernel(imputs, table):
    b, h = imputs.shape
    n_rows = b * h
    per_w = n_rows // NW
    idx3 = imputs.reshape(NW, per_w // CH, CH).astype(jnp.int32)
    out = _gather(idx3, table, n_rows=n_rows)
    return out.reshape(b, h, D)
